# baseline (speedup 1.0000x reference)
import functools

import jax
import jax.numpy as jnp
from jax import lax
from jax.experimental import pallas as pl
from jax.experimental.pallas import tpu as pltpu
from jax.experimental.pallas import tpu_sc as plsc

B = 4096
L = 200
VEC = 64
OUT = 10
DP = 16
NC = 2
NS = 16
NW = NC * NS
BPW = B // NW
IPW = BPW * L


def _proj_body(t_ref, w_ref, o_ref):
    o_ref[...] = jnp.dot(
        t_ref[...].astype(jnp.bfloat16),
        w_ref[...].astype(jnp.bfloat16),
        preferred_element_type=jnp.float32,
    )


def _project(table, w_pad):
    v = table.shape[0]
    blk = 4096
    return pl.pallas_call(
        _proj_body,
        grid=(v // blk,),
        in_specs=[
            pl.BlockSpec((blk, VEC), lambda i: (i, 0)),
            pl.BlockSpec((VEC, DP), lambda i: (0, 0)),
        ],
        out_specs=pl.BlockSpec((blk, DP), lambda i: (i, 0)),
        out_shape=jax.ShapeDtypeStruct((v, DP), jnp.float32),
        compiler_params=pltpu.CompilerParams(
            dimension_semantics=("arbitrary",),
        ),
    )(table, w_pad)


def _pool_body(p_hbm, xf_hbm, out_hbm, idx_v, rows_v, pooled_v, sem):
    wid = lax.axis_index("s") * NC + lax.axis_index("c")
    pltpu.sync_copy(xf_hbm.at[pl.ds(wid * IPW, IPW)], idx_v)

    @pl.loop(0, BPW)
    def _(lr):
        base = lr * L
        c1 = pltpu.async_copy(
            p_hbm.at[idx_v.at[pl.ds(base, 128)]],
            rows_v.at[pl.ds(0, 128)], sem)
        c2 = pltpu.async_copy(
            p_hbm.at[idx_v.at[pl.ds(base + 128, L - 128)]],
            rows_v.at[pl.ds(128, L - 128)], sem)
        c1.wait()
        c2.wait()

        def body(r, acc):
            return acc + rows_v[r, :]

        acc = lax.fori_loop(0, L, body, jnp.zeros((DP,), jnp.float32))
        pooled_v[lr, :] = acc

    pltpu.sync_copy(pooled_v, out_hbm.at[pl.ds(wid * BPW, BPW)])


def _pool(p, xf):
    mesh = plsc.VectorSubcoreMesh(core_axis_name="c", subcore_axis_name="s")
    f = pl.kernel(
        _pool_body,
        out_type=jax.ShapeDtypeStruct((B, DP), jnp.float32),
        mesh=mesh,
        scratch_types=[
            pltpu.VMEM((IPW,), jnp.int32),
            pltpu.VMEM((L, DP), jnp.float32),
            pltpu.VMEM((BPW, DP), jnp.float32),
            pltpu.SemaphoreType.DMA,
        ],
        compiler_params=pltpu.CompilerParams(use_tc_tiling_on_sc=False),
    )
    return f(p, xf)


def _head_body(p_ref, b_ref, o_ref):
    logits = p_ref[:, :OUT] * (1.0 / L) + b_ref[:, :OUT]
    m = jnp.max(logits, axis=-1, keepdims=True)
    e = jnp.exp(logits - m)
    o_ref[...] = e / jnp.sum(e, axis=-1, keepdims=True)


def _head(pooled, b_pad):
    return pl.pallas_call(
        _head_body,
        out_shape=jax.ShapeDtypeStruct((B, OUT), jnp.float32),
    )(pooled, b_pad)


def kernel(x, table, W, b):
    w_pad = jnp.zeros((VEC, DP), jnp.float32).at[:, :OUT].set(W)
    b_pad = jnp.zeros((1, DP), jnp.float32).at[0, :OUT].set(b)
    p = _project(table, w_pad)
    pooled = _pool(p, x.reshape(-1).astype(jnp.int32))
    return _head(pooled, b_pad)


# speedup vs baseline: 3.0748x; 3.0748x over previous
import jax
import jax.numpy as jnp
from jax import lax
from jax.experimental import pallas as pl
from jax.experimental.pallas import tpu as pltpu
from jax.experimental.pallas import tpu_sc as plsc

B = 4096
L = 200
VEC = 64
OUT = 10
DP = 16
NC = 2
NS = 16
NW = NC * NS
BPW = B // NW
IPW = BPW * L
G1 = 128
G2 = L - G1


BN = 8192


def _proj_body(tt_ref, w_ref, o_ref):
    q = lax.dot_general(
        tt_ref[...].astype(jnp.bfloat16),
        w_ref[...].astype(jnp.bfloat16),
        (((0,), (0,)), ((), ())),
        preferred_element_type=jnp.float32,
    )
    q3 = q.reshape(BN // 8, 8, DP)
    for s in range(8):
        o_ref[:, DP * s:DP * (s + 1)] = q3[:, s, :]


def _project(tt, w_pad):
    v = tt.shape[1]
    return pl.pallas_call(
        _proj_body,
        grid=(pl.cdiv(v, BN),),
        in_specs=[
            pl.BlockSpec((VEC, BN), lambda i: (0, i)),
            pl.BlockSpec((VEC, DP), lambda i: (0, 0)),
        ],
        out_specs=pl.BlockSpec((BN // 8, 8 * DP), lambda i: (i, 0)),
        out_shape=jax.ShapeDtypeStruct((v // 8, 8 * DP), jnp.float32),
        compiler_params=pltpu.CompilerParams(
            dimension_semantics=("arbitrary",),
        ),
    )(tt, w_pad)


def _pool_body(p_hbm, xf_hbm, out_hbm, idx_v, buf0, buf1, pooled_v,
               sem0, sem1):
    wid = lax.axis_index("s") * NC + lax.axis_index("c")
    pltpu.sync_copy(xf_hbm.at[pl.ds(wid * IPW, IPW)], idx_v)

    def start_row(buf, sem, lr):
        base = lr * L
        pltpu.make_async_copy(
            p_hbm.at[idx_v.at[pl.ds(base, G1)]],
            buf.at[pl.ds(0, G1)], sem).start()
        pltpu.make_async_copy(
            p_hbm.at[idx_v.at[pl.ds(base + G1, G2)]],
            buf.at[pl.ds(G1, G2)], sem).start()

    def wait_row(buf, sem):
        pltpu.make_async_copy(
            p_hbm.at[idx_v.at[pl.ds(0, G1)]],
            buf.at[pl.ds(0, G1)], sem).wait()
        pltpu.make_async_copy(
            p_hbm.at[idx_v.at[pl.ds(0, G2)]],
            buf.at[pl.ds(G1, G2)], sem).wait()

    def accum(buf, lr):
        def body(r, acc):
            return acc + buf[r, :]

        acc = lax.fori_loop(0, L, body, jnp.zeros((DP,), jnp.float32))
        pooled_v[lr, :] = acc

    start_row(buf0, sem0, 0)

    @pl.loop(0, BPW, step=2)
    def _(lr):
        start_row(buf1, sem1, lr + 1)
        wait_row(buf0, sem0)
        accum(buf0, lr)

        @pl.when(lr + 2 < BPW)
        def _():
            start_row(buf0, sem0, lr + 2)

        wait_row(buf1, sem1)
        accum(buf1, lr + 1)

    pltpu.sync_copy(pooled_v, out_hbm.at[pl.ds(wid * BPW, BPW)])


def _pool(p, xf):
    mesh = plsc.VectorSubcoreMesh(core_axis_name="c", subcore_axis_name="s")
    f = pl.kernel(
        _pool_body,
        out_type=jax.ShapeDtypeStruct((B, DP), jnp.float32),
        mesh=mesh,
        scratch_types=[
            pltpu.VMEM((IPW,), jnp.int32),
            pltpu.VMEM((L, DP), jnp.float32),
            pltpu.VMEM((L, DP), jnp.float32),
            pltpu.VMEM((BPW, DP), jnp.float32),
            pltpu.SemaphoreType.DMA,
            pltpu.SemaphoreType.DMA,
        ],
        compiler_params=pltpu.CompilerParams(use_tc_tiling_on_sc=False),
    )
    return f(p, xf)


def _head_body(p_ref, b_ref, o_ref):
    logits = p_ref[:, :OUT] * (1.0 / L) + b_ref[:, :OUT]
    m = jnp.max(logits, axis=-1, keepdims=True)
    e = jnp.exp(logits - m)
    o_ref[...] = e / jnp.sum(e, axis=-1, keepdims=True)


def _head(pooled, b_pad):
    return pl.pallas_call(
        _head_body,
        out_shape=jax.ShapeDtypeStruct((B, OUT), jnp.float32),
    )(pooled, b_pad)


def kernel(x, table, W, b):
    v = table.shape[0]
    w_pad = jnp.zeros((VEC, DP), jnp.float32).at[:, :OUT].set(W)
    b_pad = jnp.zeros((1, DP), jnp.float32).at[0, :OUT].set(b)
    p2 = _project(table.T, w_pad)
    p = p2.reshape(v, DP)
    pooled = _pool(p, x.reshape(-1).astype(jnp.int32))
    return _head(pooled, b_pad)


# speedup vs baseline: 3.3587x; 1.0923x over previous
import jax
import jax.numpy as jnp
from jax import lax
from jax.experimental import pallas as pl
from jax.experimental.pallas import tpu as pltpu
from jax.experimental.pallas import tpu_sc as plsc

B = 4096
L = 200
VEC = 64
OUT = 10
DP = 16
NC = 2
NS = 16
NW = NC * NS
BPW = B // NW
IPW = BPW * L
G1 = 128
G2 = L - G1


BN = 8192


def _proj_body(tt_ref, w_ref, o_ref):
    q = lax.dot_general(
        tt_ref[...].astype(jnp.bfloat16),
        w_ref[...].astype(jnp.bfloat16),
        (((0,), (0,)), ((), ())),
        preferred_element_type=jnp.float32,
    )
    q3 = q.reshape(BN // 8, 8, DP)
    for s in range(8):
        o_ref[:, DP * s:DP * (s + 1)] = q3[:, s, :]


def _project(tt, w_pad):
    v = tt.shape[1]
    return pl.pallas_call(
        _proj_body,
        grid=(pl.cdiv(v, BN),),
        in_specs=[
            pl.BlockSpec((VEC, BN), lambda i: (0, i)),
            pl.BlockSpec((VEC, DP), lambda i: (0, 0)),
        ],
        out_specs=pl.BlockSpec((BN // 8, 8 * DP), lambda i: (i, 0)),
        out_shape=jax.ShapeDtypeStruct((pl.cdiv(v, BN) * BN // 8, 8 * DP),
                                       jnp.float32),
        compiler_params=pltpu.CompilerParams(
            dimension_semantics=("arbitrary",),
        ),
    )(tt, w_pad)


def _pool_body(p_hbm, xf_hbm, out_hbm, idx_v, buf0, buf1, pooled_v,
               sem0, sem1):
    wid = lax.axis_index("s") * NC + lax.axis_index("c")
    pltpu.sync_copy(xf_hbm.at[pl.ds(wid * IPW, IPW)], idx_v)

    def start_row(buf, sem, lr):
        base = lr * L
        pltpu.make_async_copy(
            p_hbm.at[idx_v.at[pl.ds(base, G1)]],
            buf.at[pl.ds(0, G1)], sem).start()
        pltpu.make_async_copy(
            p_hbm.at[idx_v.at[pl.ds(base + G1, G2)]],
            buf.at[pl.ds(G1, G2)], sem).start()

    def wait_row(buf, sem):
        pltpu.make_async_copy(
            p_hbm.at[idx_v.at[pl.ds(0, G1)]],
            buf.at[pl.ds(0, G1)], sem).wait()
        pltpu.make_async_copy(
            p_hbm.at[idx_v.at[pl.ds(0, G2)]],
            buf.at[pl.ds(G1, G2)], sem).wait()

    def accum(buf, lr):
        vals = [buf[r, :] for r in range(L)]
        while len(vals) > 1:
            nxt = [vals[i] + vals[i + 1] for i in range(0, len(vals) - 1, 2)]
            if len(vals) % 2:
                nxt.append(vals[-1])
            vals = nxt
        pooled_v[lr, :] = vals[0]

    start_row(buf0, sem0, 0)

    @pl.loop(0, BPW, step=2)
    def _(lr):
        start_row(buf1, sem1, lr + 1)
        wait_row(buf0, sem0)
        accum(buf0, lr)

        @pl.when(lr + 2 < BPW)
        def _():
            start_row(buf0, sem0, lr + 2)

        wait_row(buf1, sem1)
        accum(buf1, lr + 1)

    pltpu.sync_copy(pooled_v, out_hbm.at[pl.ds(wid * BPW, BPW)])


def _pool(p, xf):
    mesh = plsc.VectorSubcoreMesh(core_axis_name="c", subcore_axis_name="s")
    f = pl.kernel(
        _pool_body,
        out_type=jax.ShapeDtypeStruct((B, DP), jnp.float32),
        mesh=mesh,
        scratch_types=[
            pltpu.VMEM((IPW,), jnp.int32),
            pltpu.VMEM((L, DP), jnp.float32),
            pltpu.VMEM((L, DP), jnp.float32),
            pltpu.VMEM((BPW, DP), jnp.float32),
            pltpu.SemaphoreType.DMA,
            pltpu.SemaphoreType.DMA,
        ],
        compiler_params=pltpu.CompilerParams(use_tc_tiling_on_sc=False),
    )
    return f(p, xf)


def _head_body(p_ref, b_ref, o_ref):
    logits = p_ref[:, :OUT] * (1.0 / L) + b_ref[:, :OUT]
    m = jnp.max(logits, axis=-1, keepdims=True)
    e = jnp.exp(logits - m)
    o_ref[...] = e / jnp.sum(e, axis=-1, keepdims=True)


def _head(pooled, b_pad):
    return pl.pallas_call(
        _head_body,
        out_shape=jax.ShapeDtypeStruct((B, OUT), jnp.float32),
    )(pooled, b_pad)


def kernel(x, table, W, b):
    v = table.shape[0]
    w_pad = jnp.zeros((VEC, DP), jnp.float32).at[:, :OUT].set(W)
    b_pad = jnp.zeros((1, DP), jnp.float32).at[0, :OUT].set(b)
    p2 = _project(table.T, w_pad)
    p = p2.reshape(p2.shape[0] * 8, DP)
    pooled = _pool(p, x.reshape(-1).astype(jnp.int32))
    return _head(pooled, b_pad)


# speedup vs baseline: 3.5334x; 1.0520x over previous
import jax
import jax.numpy as jnp
from jax import lax
from jax.experimental import pallas as pl
from jax.experimental.pallas import tpu as pltpu
from jax.experimental.pallas import tpu_sc as plsc

B = 4096
L = 200
VEC = 64
OUT = 10
DP = 16
NC = 2
NS = 16
NW = NC * NS
BPW = B // NW
IPW = BPW * L
G1 = 128
G2 = L - G1


BN = 8192


def _proj_body(tt_ref, w_ref, o_ref):
    q = lax.dot_general(
        tt_ref[...].astype(jnp.bfloat16),
        w_ref[...].astype(jnp.bfloat16),
        (((0,), (0,)), ((), ())),
        preferred_element_type=jnp.float32,
    )
    q5 = q.reshape(BN // 64, 8, 8, DP)
    o_ref[...] = jnp.concatenate(
        [q5[:, t] for t in range(8)], axis=2).reshape(BN // 8, 8 * DP)


def _project(tt, w_pad):
    v = tt.shape[1]
    return pl.pallas_call(
        _proj_body,
        grid=(pl.cdiv(v, BN),),
        in_specs=[
            pl.BlockSpec((VEC, BN), lambda i: (0, i)),
            pl.BlockSpec((VEC, DP), lambda i: (0, 0)),
        ],
        out_specs=pl.BlockSpec((BN // 8, 8 * DP), lambda i: (i, 0)),
        out_shape=jax.ShapeDtypeStruct((pl.cdiv(v, BN) * BN // 8, 8 * DP),
                                       jnp.float32),
        compiler_params=pltpu.CompilerParams(
            dimension_semantics=("arbitrary",),
        ),
    )(tt, w_pad)


def _pool_body(p_hbm, xf_hbm, out_hbm, idx_v, buf0, buf1, pooled_v,
               sem0, sem1):
    wid = lax.axis_index("s") * NC + lax.axis_index("c")
    pltpu.sync_copy(xf_hbm.at[pl.ds(wid * IPW, IPW)], idx_v)

    @pl.loop(0, IPW, step=16)
    def _(t):
        r = idx_v[pl.ds(t, 16)]
        idx_v[pl.ds(t, 16)] = (
            (r & jnp.int32(-64))
            | ((r & jnp.int32(7)) << 3)
            | ((r >> 3) & jnp.int32(7))
        )

    def start_row(buf, sem, lr):
        base = lr * L
        pltpu.make_async_copy(
            p_hbm.at[idx_v.at[pl.ds(base, G1)]],
            buf.at[pl.ds(0, G1)], sem).start()
        pltpu.make_async_copy(
            p_hbm.at[idx_v.at[pl.ds(base + G1, G2)]],
            buf.at[pl.ds(G1, G2)], sem).start()

    def wait_row(buf, sem):
        pltpu.make_async_copy(
            p_hbm.at[idx_v.at[pl.ds(0, G1)]],
            buf.at[pl.ds(0, G1)], sem).wait()
        pltpu.make_async_copy(
            p_hbm.at[idx_v.at[pl.ds(0, G2)]],
            buf.at[pl.ds(G1, G2)], sem).wait()

    def accum(buf, lr):
        vals = [buf[r, :] for r in range(L)]
        while len(vals) > 1:
            nxt = [vals[i] + vals[i + 1] for i in range(0, len(vals) - 1, 2)]
            if len(vals) % 2:
                nxt.append(vals[-1])
            vals = nxt
        pooled_v[lr, :] = vals[0]

    start_row(buf0, sem0, 0)

    @pl.loop(0, BPW, step=2)
    def _(lr):
        start_row(buf1, sem1, lr + 1)
        wait_row(buf0, sem0)
        accum(buf0, lr)

        @pl.when(lr + 2 < BPW)
        def _():
            start_row(buf0, sem0, lr + 2)

        wait_row(buf1, sem1)
        accum(buf1, lr + 1)

    pltpu.sync_copy(pooled_v, out_hbm.at[pl.ds(wid * BPW, BPW)])


def _pool(p, xf):
    mesh = plsc.VectorSubcoreMesh(core_axis_name="c", subcore_axis_name="s")
    f = pl.kernel(
        _pool_body,
        out_type=jax.ShapeDtypeStruct((B, DP), jnp.float32),
        mesh=mesh,
        scratch_types=[
            pltpu.VMEM((IPW,), jnp.int32),
            pltpu.VMEM((L, DP), jnp.float32),
            pltpu.VMEM((L, DP), jnp.float32),
            pltpu.VMEM((BPW, DP), jnp.float32),
            pltpu.SemaphoreType.DMA,
            pltpu.SemaphoreType.DMA,
        ],
        compiler_params=pltpu.CompilerParams(use_tc_tiling_on_sc=False),
    )
    return f(p, xf)


def _head_body(p_ref, b_ref, o_ref):
    logits = p_ref[:, :OUT] * (1.0 / L) + b_ref[:, :OUT]
    m = jnp.max(logits, axis=-1, keepdims=True)
    e = jnp.exp(logits - m)
    o_ref[...] = e / jnp.sum(e, axis=-1, keepdims=True)


def _head(pooled, b_pad):
    return pl.pallas_call(
        _head_body,
        out_shape=jax.ShapeDtypeStruct((B, OUT), jnp.float32),
    )(pooled, b_pad)


def kernel(x, table, W, b):
    v = table.shape[0]
    w_pad = jnp.zeros((VEC, DP), jnp.float32).at[:, :OUT].set(W)
    b_pad = jnp.zeros((1, DP), jnp.float32).at[0, :OUT].set(b)
    p2 = _project(table.T, w_pad)
    p = p2.reshape(p2.shape[0] * 8, DP)
    pooled = _pool(p, x.reshape(-1).astype(jnp.int32))
    return _head(pooled, b_pad)
